# baseline (speedup 1.0000x reference)
import functools

import jax
import jax.numpy as jnp
from jax import lax
from jax.experimental import pallas as pl
from jax.experimental.pallas import tpu as pltpu
from jax.experimental.pallas import tpu_sc as plsc

_E = 64
_TOPK = 8
_BLK = 1024
_T = 16384
_NCHUNK = 4
_CT = _T // _NCHUNK


def _p_kernel(hs_ref, w_ref, p_ref):
    hs = hs_ref[...]
    w = w_ref[...]
    logits = lax.dot_general(w, hs, (((1,), (1,)), ((), ())),
                             preferred_element_type=jnp.float32)
    colmax = jnp.max(logits, axis=0, keepdims=True)
    p_ref[...] = jnp.exp(logits - colmax)


def _tc_scores_t(hs, w, c):
    t, h = hs.shape
    steps = _CT // _BLK
    return pl.pallas_call(
        _p_kernel,
        grid=(steps,),
        in_specs=[
            pl.BlockSpec((_BLK, h), lambda i, c=c: (i + c * steps, 0)),
            pl.BlockSpec((_E, h), lambda i: (0, 0)),
        ],
        out_specs=pl.BlockSpec((_E, _BLK), lambda i: (0, i)),
        out_shape=jax.ShapeDtypeStruct((_E, _CT), jnp.float32),
        compiler_params=pltpu.CompilerParams(
            dimension_semantics=("parallel",)),
    )(hs, w)


def _make_sc_topk():
    info = plsc.get_sparse_core_info()
    nc, ns = info.num_cores, info.num_subcores
    nw = nc * ns
    chunk = _CT // nw
    ngroups = chunk // 16
    mesh = plsc.VectorSubcoreMesh(core_axis_name="c", subcore_axis_name="s")

    @functools.partial(
        pl.kernel,
        mesh=mesh,
        out_type=[
            jax.ShapeDtypeStruct((_TOPK, _CT), jnp.int32),
            jax.ShapeDtypeStruct((_TOPK, _CT), jnp.float32),
        ],
        scratch_types=[
            pltpu.VMEM((_E, chunk), jnp.float32),
            pltpu.VMEM((_TOPK, chunk), jnp.int32),
            pltpu.VMEM((_TOPK, chunk), jnp.float32),
        ],
    )
    def sc_topk(p_hbm, idx_hbm, w_hbm, p_v, idx_v, w_v):
        wid = lax.axis_index("s") * nc + lax.axis_index("c")
        base = wid * chunk
        pltpu.sync_copy(p_hbm.at[:, pl.ds(base, chunk)], p_v)

        def group_body(g, _):
            col = g * 16

            vs = [jnp.full((16,), -1.0, jnp.float32) for _ in range(_TOPK)]
            ids = [jnp.zeros((16,), jnp.int32) for _ in range(_TOPK)]
            for e in range(_E):
                val = p_v[e, pl.ds(col, 16)]
                vid = jnp.full((16,), e, jnp.int32)
                for j in range(_TOPK):
                    swap = val > vs[j]
                    nv = jnp.where(swap, val, vs[j])
                    val = jnp.where(swap, vs[j], val)
                    ni = jnp.where(swap, vid, ids[j])
                    vid = jnp.where(swap, ids[j], vid)
                    vs[j] = nv
                    ids[j] = ni
            denom = vs[0]
            for j in range(1, _TOPK):
                denom = denom + vs[j]
            denom = denom + 1e-20
            for j in range(_TOPK):
                idx_v[j, pl.ds(col, 16)] = ids[j]
                w_v[j, pl.ds(col, 16)] = vs[j] / denom
            return 0

        lax.fori_loop(0, ngroups, group_body, 0)
        pltpu.sync_copy(idx_v, idx_hbm.at[:, pl.ds(base, chunk)])
        pltpu.sync_copy(w_v, w_hbm.at[:, pl.ds(base, chunk)])

    return sc_topk


def kernel(hidden_states, weight):
    bsz, seq, h = hidden_states.shape
    t = bsz * seq
    hs = hidden_states.reshape(t, h)
    sc_topk = _make_sc_topk()
    idxs = []
    ws = []
    for c in range(_NCHUNK):
        p_t = _tc_scores_t(hs, weight, c)
        idx_t, w_t = sc_topk(p_t)
        idxs.append(idx_t)
        ws.append(w_t)
    idx = jnp.concatenate(idxs, axis=1).T
    w = jnp.concatenate(ws, axis=1).T
    return (idx, w)


# speedup vs baseline: 1.2897x; 1.2897x over previous
import jax
import jax.numpy as jnp
from jax.experimental import pallas as pl
from jax.experimental.pallas import tpu as pltpu

_E = 64
_TOPK = 8
_BLK = 1024


def _gate_kernel(hs_ref, wt_ref, idx_ref, w_ref):
    hs = hs_ref[...]
    wt = wt_ref[...]
    logits = jnp.dot(hs, wt, preferred_element_type=jnp.float32)
    rowmax = jnp.max(logits, axis=-1, keepdims=True)
    p = jnp.exp(logits - rowmax)
    b = p.shape[0]
    iota = jax.lax.broadcasted_iota(jnp.int32, (b, _E), 1)
    bits = jax.lax.bitcast_convert_type(p, jnp.int32)
    enc = ((bits & ~0x3F) | (_E - 1 - iota)) + 0x3F800000
    encf = jax.lax.bitcast_convert_type(enc, jnp.float32)
    vals = []
    keys = []
    for _ in range(_TOPK):
        mv = jnp.max(p, axis=-1, keepdims=True)
        cand = jnp.where(p == mv, encf, 0.0)
        m2 = jnp.max(cand, axis=-1, keepdims=True)
        keys.append(m2)
        vals.append(mv)
        kill = encf == m2
        p = jnp.where(kill, -1.0, p)
        encf = jnp.where(kill, 0.0, encf)
    v = jnp.concatenate(vals, axis=-1)
    kbits = jax.lax.bitcast_convert_type(
        jnp.concatenate(keys, axis=-1), jnp.int32)
    i = (_E - 1) - (kbits & 0x3F)
    denom = jnp.sum(v, axis=-1, keepdims=True) + 1e-20
    idx_ref[...] = i.T
    w_ref[...] = (v / denom).T


def kernel(hidden_states, weight):
    bsz, seq, h = hidden_states.shape
    t = bsz * seq
    hs = hidden_states.reshape(t, h)
    wt = weight.T

    idx_t, w_t = pl.pallas_call(
        _gate_kernel,
        grid=(t // _BLK,),
        in_specs=[
            pl.BlockSpec((_BLK, h), lambda i: (i, 0)),
            pl.BlockSpec((h, _E), lambda i: (0, 0)),
        ],
        out_specs=[
            pl.BlockSpec((_TOPK, _BLK), lambda i: (0, i)),
            pl.BlockSpec((_TOPK, _BLK), lambda i: (0, i)),
        ],
        out_shape=[
            jax.ShapeDtypeStruct((_TOPK, t), jnp.int32),
            jax.ShapeDtypeStruct((_TOPK, t), jnp.float32),
        ],
        compiler_params=pltpu.CompilerParams(
            dimension_semantics=("parallel",)),
    )(hs, wt)

    return (idx_t.T, w_t.T)
